# Initial kernel scaffold; baseline (speedup 1.0000x reference)
import functools

import jax
import jax.numpy as jnp
from jax import lax
from jax.experimental import pallas as pl
from jax.experimental.pallas import tpu as pltpu
from jax.experimental.pallas import tpu_sc as plsc

X = 512
Y = 512
NCELL = X * Y
B, V, C = 2, 20000, 64
VPAD = 20008
L = 16
NC, NS = 2, 16
NW = NC * NS
SLAB = NCELL // NW
CHUNK = 8192

_mesh = plsc.VectorSubcoreMesh(
    core_axis_name="c", subcore_axis_name="s", num_cores=NC, num_subcores=NS
)


def _transpose_body(f_ref, o_ref):
    o_ref[...] = f_ref[...].T


def _transpose(feat_pad):
    return pl.pallas_call(
        _transpose_body,
        grid=(B,),
        in_specs=[pl.BlockSpec((None, VPAD, C), lambda b: (b, 0, 0))],
        out_specs=pl.BlockSpec((None, C, VPAD), lambda b: (b, 0, 0)),
        out_shape=jax.ShapeDtypeStruct((B, C, VPAD), jnp.float32),
    )(feat_pad)


@functools.partial(
    pl.kernel,
    out_type=jax.ShapeDtypeStruct((B, NCELL), jnp.int32),
    mesh=_mesh,
    scratch_types=[
        pltpu.VMEM((V, 3), jnp.int32),
        pltpu.VMEM((SLAB,), jnp.int32),
    ],
)
def _winner_kernel(coords_hbm, winner_hbm, cbuf, wslab):
    wid = lax.axis_index("s") * NC + lax.axis_index("c")
    base = wid * SLAB
    lanes = lax.iota(jnp.int32, L)
    zeros16 = jnp.zeros((L,), jnp.int32)
    ones16 = jnp.ones((L,), jnp.int32)

    for b in range(B):
        pltpu.sync_copy(coords_hbm.at[b], cbuf)

        def initf(j, carry):
            wslab[pl.ds(j * L, L)] = jnp.full((L,), -1, jnp.int32)
            return carry

        lax.fori_loop(0, SLAB // L, initf, 0)

        def grp(i, carry):
            vidx = i * L + lanes
            xs = plsc.load_gather(cbuf, [vidx, zeros16])
            ys = plsc.load_gather(cbuf, [vidx, ones16])
            li = ys * X + xs - base
            m = (li >= 0) & (li < SLAB)
            li_safe = jnp.clip(li, 0, SLAB - 1)

            @pl.when(jnp.any(m))
            def _():
                def body(_):
                    cur = plsc.load_gather(wslab, [li_safe])
                    need = m & (cur < vidx)
                    plsc.store_scatter(
                        wslab, [li_safe], jnp.maximum(cur, vidx), mask=need
                    )
                    return jnp.any(need)

                lax.while_loop(lambda c: c, body, jnp.bool_(True))

            return carry

        lax.fori_loop(0, V // L, grp, 0)
        pltpu.sync_copy(wslab, winner_hbm.at[b, pl.ds(base, SLAB)])


@functools.partial(
    pl.kernel,
    out_type=jax.ShapeDtypeStruct((B, C, NCELL), jnp.float32),
    mesh=_mesh,
    scratch_types=[
        pltpu.VMEM((VPAD,), jnp.float32),
        pltpu.VMEM((VPAD,), jnp.float32),
        pltpu.VMEM((CHUNK,), jnp.int32),
        pltpu.VMEM((CHUNK,), jnp.float32),
        pltpu.VMEM((CHUNK,), jnp.float32),
    ],
)
def _emit_kernel(featT_hbm, winner_hbm, out_hbm, ft0, ft1, wbuf, obuf0, obuf1):
    wid = lax.axis_index("s") * NC + lax.axis_index("c")
    c0 = wid * 2

    for b in range(B):
        pltpu.sync_copy(featT_hbm.at[b, c0], ft0)
        pltpu.sync_copy(featT_hbm.at[b, c0 + 1], ft1)

        def chunk_loop(k, carry):
            off = k * CHUNK
            pltpu.sync_copy(winner_hbm.at[b, pl.ds(off, CHUNK)], wbuf)

            def grp(j, c2):
                w = wbuf[pl.ds(j * L, L)]
                g = jnp.where(w >= 0, w, V)
                obuf0[pl.ds(j * L, L)] = plsc.load_gather(ft0, [g])
                obuf1[pl.ds(j * L, L)] = plsc.load_gather(ft1, [g])
                return c2

            lax.fori_loop(0, CHUNK // L, grp, 0)
            pltpu.sync_copy(obuf0, out_hbm.at[b, c0, pl.ds(off, CHUNK)])
            pltpu.sync_copy(obuf1, out_hbm.at[b, c0 + 1, pl.ds(off, CHUNK)])
            return carry

        lax.fori_loop(0, NCELL // CHUNK, chunk_loop, 0)


def kernel(pillar_features, coords):
    feat_pad = jnp.pad(pillar_features, ((0, 0), (0, VPAD - V), (0, 0)))
    featT = _transpose(feat_pad)
    winner = _winner_kernel(coords)
    out = _emit_kernel(featT, winner)
    return out.reshape(B, C, Y, X)


# speedup vs baseline: 34.0106x; 34.0106x over previous
import functools

import jax
import jax.numpy as jnp
from jax import lax
from jax.experimental import pallas as pl
from jax.experimental.pallas import tpu as pltpu
from jax.experimental.pallas import tpu_sc as plsc

X = 512
Y = 512
NCELL = X * Y
B, V, C = 2, 20000, 64
VPAD = 20008
L = 16
NC, NS = 2, 16
NW = NC * NS
SLAB = NCELL // NW
CHUNK = 8192

_mesh = plsc.VectorSubcoreMesh(
    core_axis_name="c", subcore_axis_name="s", num_cores=NC, num_subcores=NS
)
_sc_params = pltpu.CompilerParams(needs_layout_passes=False)


def _transpose_body(f_ref, o_ref):
    o_ref[...] = f_ref[...].T


def _transpose(feat_pad):
    return pl.pallas_call(
        _transpose_body,
        grid=(B,),
        in_specs=[pl.BlockSpec((None, VPAD, C), lambda b: (b, 0, 0))],
        out_specs=pl.BlockSpec((None, C, VPAD), lambda b: (b, 0, 0)),
        out_shape=jax.ShapeDtypeStruct((B, C, VPAD), jnp.float32),
    )(feat_pad)


@functools.partial(
    pl.kernel,
    out_type=jax.ShapeDtypeStruct((B, NCELL), jnp.int32),
    mesh=_mesh,
    compiler_params=_sc_params,
    scratch_types=[
        pltpu.VMEM((V * 3,), jnp.int32),
        pltpu.VMEM((SLAB,), jnp.int32),
    ],
)
def _winner_kernel(coords_hbm, winner_hbm, cbuf, wslab):
    wid = lax.axis_index("s") * NC + lax.axis_index("c")
    base = wid * SLAB
    lanes = lax.iota(jnp.int32, L)

    for b in range(B):
        pltpu.sync_copy(coords_hbm.at[b], cbuf)

        def initf(j, carry):
            wslab[pl.ds(j * L, L)] = jnp.full((L,), -1, jnp.int32)
            return carry

        lax.fori_loop(0, SLAB // L, initf, 0)

        def grp(i, carry):
            vidx = i * L + lanes
            vidx3 = vidx * 3
            xs = plsc.load_gather(cbuf, [vidx3])
            ys = plsc.load_gather(cbuf, [vidx3 + 1])
            li = ys * X + xs - base
            m = (li >= 0) & (li < SLAB)
            li_safe = jnp.clip(li, 0, SLAB - 1)

            @pl.when(jnp.any(m))
            def _():
                def body(_):
                    cur = plsc.load_gather(wslab, [li_safe])
                    need = m & (cur < vidx)
                    plsc.store_scatter(
                        wslab, [li_safe], jnp.maximum(cur, vidx), mask=need
                    )
                    return jnp.any(need)

                lax.while_loop(lambda c: c, body, jnp.bool_(True))

            return carry

        lax.fori_loop(0, V // L, grp, 0)
        pltpu.sync_copy(wslab, winner_hbm.at[b, pl.ds(base, SLAB)])


@functools.partial(
    pl.kernel,
    out_type=jax.ShapeDtypeStruct((B, C, NCELL), jnp.float32),
    mesh=_mesh,
    compiler_params=_sc_params,
    scratch_types=[
        pltpu.VMEM((VPAD,), jnp.float32),
        pltpu.VMEM((VPAD,), jnp.float32),
        pltpu.VMEM((CHUNK,), jnp.int32),
        pltpu.VMEM((CHUNK,), jnp.float32),
        pltpu.VMEM((CHUNK,), jnp.float32),
    ],
)
def _emit_kernel(featT_hbm, winner_hbm, out_hbm, ft0, ft1, wbuf, obuf0, obuf1):
    wid = lax.axis_index("s") * NC + lax.axis_index("c")
    c0 = wid * 2

    for b in range(B):
        pltpu.sync_copy(featT_hbm.at[b, c0], ft0)
        pltpu.sync_copy(featT_hbm.at[b, c0 + 1], ft1)

        def chunk_loop(k, carry):
            off = k * CHUNK
            pltpu.sync_copy(winner_hbm.at[b, pl.ds(off, CHUNK)], wbuf)

            def grp(j, c2):
                w = wbuf[pl.ds(j * L, L)]
                g = jnp.where(w >= 0, w, V)
                obuf0[pl.ds(j * L, L)] = plsc.load_gather(ft0, [g])
                obuf1[pl.ds(j * L, L)] = plsc.load_gather(ft1, [g])
                return c2

            lax.fori_loop(0, CHUNK // L, grp, 0)
            pltpu.sync_copy(obuf0, out_hbm.at[b, c0, pl.ds(off, CHUNK)])
            pltpu.sync_copy(obuf1, out_hbm.at[b, c0 + 1, pl.ds(off, CHUNK)])
            return carry

        lax.fori_loop(0, NCELL // CHUNK, chunk_loop, 0)


def kernel(pillar_features, coords):
    feat_pad = jnp.pad(pillar_features, ((0, 0), (0, VPAD - V), (0, 0)))
    featT = _transpose(feat_pad)
    winner = _winner_kernel(coords.reshape(B, V * 3))
    out = _emit_kernel(featT, winner)
    return out.reshape(B, C, Y, X)
